# Initial kernel scaffold; baseline (speedup 1.0000x reference)
import jax
import jax.numpy as jnp
from jax.experimental import pallas as pl


def kernel(weight_q, scale, grid):
    raise NotImplementedError("write your pallas kernel here")


# speedup vs baseline: 33.2229x; 33.2229x over previous
import functools

import jax
import jax.numpy as jnp
from jax import lax
from jax.experimental import pallas as pl
from jax.experimental.pallas import tpu as pltpu
from jax.experimental.pallas import tpu_sc as plsc

OUT_F = 4096
IN_F = 4096
CODESZ = 8
GRID_K = 65536

NC = 2
NS = 16
NW = NC * NS

TOTAL_IDX = OUT_F * (IN_F // CODESZ)
IDX_PER_W = TOTAL_IDX // NW
CHUNK = 4096
N_CHUNKS = IDX_PER_W // CHUNK


def _scale_body(s_ref, g_ref, o_ref):
    o_ref[...] = g_ref[...] * s_ref[0]


@jax.jit
def _scale_table(scale, grid_rs):
    return pl.pallas_call(
        _scale_body,
        out_shape=jax.ShapeDtypeStruct(grid_rs.shape, jnp.float32),
        in_specs=[
            pl.BlockSpec(memory_space=pltpu.SMEM),
            pl.BlockSpec(memory_space=pltpu.VMEM),
        ],
        out_specs=pl.BlockSpec(memory_space=pltpu.VMEM),
    )(scale, grid_rs)


_MESH = plsc.VectorSubcoreMesh(
    core_axis_name="c", subcore_axis_name="s", num_cores=NC, num_subcores=NS
)


@functools.partial(
    pl.kernel,
    out_type=jax.ShapeDtypeStruct((TOTAL_IDX, CODESZ), jnp.float32),
    mesh=_MESH,
    scratch_types=[
        pltpu.VMEM((CHUNK,), jnp.int32),
        pltpu.VMEM((CHUNK, CODESZ), jnp.float32),
        pltpu.SemaphoreType.DMA,
    ],
    compiler_params=pltpu.CompilerParams(use_tc_tiling_on_sc=False),
)
def _sc_gather(tab_hbm, wq_hbm, out_hbm, idx_v, rows_v, sem):
    wid = lax.axis_index("s") * NC + lax.axis_index("c")
    base0 = wid * IDX_PER_W

    def body(k, carry):
        base = base0 + k * CHUNK
        pltpu.sync_copy(wq_hbm.at[pl.ds(base, CHUNK)], idx_v)
        pltpu.async_copy(tab_hbm.at[idx_v], rows_v, sem).wait()
        pltpu.sync_copy(rows_v, out_hbm.at[pl.ds(base, CHUNK)])
        return carry

    lax.fori_loop(0, N_CHUNKS, body, 0)


def kernel(weight_q, scale, grid):
    wq = weight_q.astype(jnp.int32).reshape(-1)
    grid_rs = grid.reshape(GRID_K * CODESZ // 128, 128)
    tab = _scale_table(scale, grid_rs).reshape(GRID_K, CODESZ)
    out = _sc_gather(tab, wq)
    return out.reshape(OUT_F, IN_F)


# speedup vs baseline: 37.4287x; 1.1266x over previous
import functools

import jax
import jax.numpy as jnp
from jax import lax
from jax.experimental import pallas as pl
from jax.experimental.pallas import tpu as pltpu
from jax.experimental.pallas import tpu_sc as plsc

OUT_F = 4096
IN_F = 4096
CODESZ = 8
GRID_K = 65536

NC = 2
NS = 16
NW = NC * NS

TOTAL_IDX = OUT_F * (IN_F // CODESZ)
IDX_PER_W = TOTAL_IDX // NW
CHUNK = 4096
N_CHUNKS = IDX_PER_W // CHUNK


def _scale_body(s_ref, g_ref, o_ref):
    o_ref[...] = g_ref[...] * s_ref[0]


@jax.jit
def _scale_table(scale, grid_rs):
    return pl.pallas_call(
        _scale_body,
        out_shape=jax.ShapeDtypeStruct(grid_rs.shape, jnp.float32),
        in_specs=[
            pl.BlockSpec(memory_space=pltpu.SMEM),
            pl.BlockSpec(memory_space=pltpu.VMEM),
        ],
        out_specs=pl.BlockSpec(memory_space=pltpu.VMEM),
    )(scale, grid_rs)


_MESH = plsc.VectorSubcoreMesh(
    core_axis_name="c", subcore_axis_name="s", num_cores=NC, num_subcores=NS
)


@functools.partial(
    pl.kernel,
    out_type=jax.ShapeDtypeStruct((TOTAL_IDX, CODESZ), jnp.float32),
    mesh=_MESH,
    scratch_types=[
        pltpu.VMEM((2, CHUNK), jnp.int32),
        pltpu.VMEM((2, CHUNK, CODESZ), jnp.float32),
        pltpu.SemaphoreType.DMA,
        pltpu.SemaphoreType.DMA((2,)),
        pltpu.SemaphoreType.DMA((2,)),
    ],
    compiler_params=pltpu.CompilerParams(use_tc_tiling_on_sc=False),
)
def _sc_gather(tab_hbm, wq_hbm, out_hbm, idx_v, rows_v, sem_i, sem_g, sem_o):
    wid = lax.axis_index("s") * NC + lax.axis_index("c")
    base0 = wid * IDX_PER_W

    def idx_src(k):
        return wq_hbm.at[pl.ds(base0 + k * CHUNK, CHUNK)]

    def out_dst(k):
        return out_hbm.at[pl.ds(base0 + k * CHUNK, CHUNK)]

    pltpu.async_copy(idx_src(0), idx_v.at[0], sem_i)
    pltpu.async_copy(idx_src(1), idx_v.at[1], sem_i)
    pltpu.make_async_copy(idx_src(0), idx_v.at[0], sem_i).wait()
    pltpu.async_copy(tab_hbm.at[idx_v.at[0]], rows_v.at[0], sem_g.at[0])

    def body(k0, carry):
        for b in range(2):
            k = 2 * k0 + b
            b1 = 1 - b

            @pl.when(k + 1 < N_CHUNKS)
            def _():
                pltpu.make_async_copy(idx_src(0), idx_v.at[b1], sem_i).wait()

                @pl.when(k >= 1)
                def _():
                    pltpu.make_async_copy(
                        rows_v.at[b1], out_dst(0), sem_o.at[b1]).wait()

                pltpu.async_copy(
                    tab_hbm.at[idx_v.at[b1]], rows_v.at[b1], sem_g.at[b1])

            pltpu.make_async_copy(
                tab_hbm.at[idx_v.at[b]], rows_v.at[b], sem_g.at[b]).wait()

            @pl.when(k + 2 < N_CHUNKS)
            def _():
                pltpu.async_copy(idx_src(k + 2), idx_v.at[b], sem_i)

            pltpu.async_copy(rows_v.at[b], out_dst(k), sem_o.at[b])
        return carry

    lax.fori_loop(0, N_CHUNKS // 2, body, 0)

    for b in range(2):
        pltpu.make_async_copy(rows_v.at[b], out_dst(0), sem_o.at[b]).wait()


def kernel(weight_q, scale, grid):
    wq = weight_q.astype(jnp.int32).reshape(-1)
    grid_rs = grid.reshape(GRID_K * CODESZ // 128, 128)
    tab = _scale_table(scale, grid_rs).reshape(GRID_K, CODESZ)
    out = _sc_gather(tab, wq)
    return out.reshape(OUT_F, IN_F)


# speedup vs baseline: 41.3162x; 1.1039x over previous
import functools

import jax
import jax.numpy as jnp
from jax import lax
from jax.experimental import pallas as pl
from jax.experimental.pallas import tpu as pltpu
from jax.experimental.pallas import tpu_sc as plsc

OUT_F = 4096
IN_F = 4096
CODESZ = 8
GRID_K = 65536

NC = 2
NS = 16
NW = NC * NS

TOTAL_IDX = OUT_F * (IN_F // CODESZ)
NQ = 4
Q_IDX = TOTAL_IDX // NQ
IDX_PER_W = Q_IDX // NW
CHUNK = 4096
N_CHUNKS = IDX_PER_W // CHUNK


def _scale_body(s_ref, g_ref, o_ref):
    o_ref[...] = g_ref[...] * s_ref[0]


def _scale_table(scale, grid_rs):
    return pl.pallas_call(
        _scale_body,
        out_shape=jax.ShapeDtypeStruct(grid_rs.shape, jnp.float32),
        in_specs=[
            pl.BlockSpec(memory_space=pltpu.SMEM),
            pl.BlockSpec(memory_space=pltpu.VMEM),
        ],
        out_specs=pl.BlockSpec(memory_space=pltpu.VMEM),
    )(scale, grid_rs)


_MESH = plsc.VectorSubcoreMesh(
    core_axis_name="c", subcore_axis_name="s", num_cores=NC, num_subcores=NS
)


def _make_gather(q):
    @functools.partial(
        pl.kernel,
        out_type=jax.ShapeDtypeStruct((Q_IDX, CODESZ), jnp.float32),
        mesh=_MESH,
        scratch_types=[
            pltpu.VMEM((2, CHUNK), jnp.int32),
            pltpu.VMEM((2, CHUNK, CODESZ), jnp.float32),
            pltpu.VMEM_SHARED((GRID_K, CODESZ), jnp.float32),
            pltpu.SemaphoreType.DMA,
            pltpu.SemaphoreType.DMA((2,)),
            pltpu.SemaphoreType.DMA((2,)),
        ],
        compiler_params=pltpu.CompilerParams(use_tc_tiling_on_sc=False),
    )
    def _sc_gather(tab_hbm, wq_hbm, out_hbm, idx_v, rows_v, tab_sp,
                   sem_i, sem_g, sem_o):
        s = lax.axis_index("s")
        wid = s * NC + lax.axis_index("c")
        base0 = q * Q_IDX + wid * IDX_PER_W
        obase0 = wid * IDX_PER_W

        tr = GRID_K // NS
        pltpu.sync_copy(tab_hbm.at[pl.ds(s * tr, tr)],
                        tab_sp.at[pl.ds(s * tr, tr)])
        plsc.subcore_barrier()

        def idx_src(k):
            return wq_hbm.at[pl.ds(base0 + k * CHUNK, CHUNK)]

        def out_dst(k):
            return out_hbm.at[pl.ds(obase0 + k * CHUNK, CHUNK)]

        pltpu.async_copy(idx_src(0), idx_v.at[0], sem_i)
        pltpu.async_copy(idx_src(1), idx_v.at[1], sem_i)
        pltpu.make_async_copy(idx_src(0), idx_v.at[0], sem_i).wait()
        pltpu.async_copy(tab_sp.at[idx_v.at[0]], rows_v.at[0], sem_g.at[0])

        def body(k0, carry):
            for b in range(2):
                k = 2 * k0 + b
                b1 = 1 - b

                @pl.when(k + 1 < N_CHUNKS)
                def _():
                    pltpu.make_async_copy(
                        idx_src(0), idx_v.at[b1], sem_i).wait()

                    @pl.when(k >= 1)
                    def _():
                        pltpu.make_async_copy(
                            rows_v.at[b1], out_dst(0), sem_o.at[b1]).wait()

                    pltpu.async_copy(
                        tab_sp.at[idx_v.at[b1]], rows_v.at[b1], sem_g.at[b1])

                pltpu.make_async_copy(
                    tab_sp.at[idx_v.at[b]], rows_v.at[b], sem_g.at[b]).wait()

                @pl.when(k + 2 < N_CHUNKS)
                def _():
                    pltpu.async_copy(idx_src(k + 2), idx_v.at[b], sem_i)

                pltpu.async_copy(rows_v.at[b], out_dst(k), sem_o.at[b])
            return carry

        lax.fori_loop(0, N_CHUNKS // 2, body, 0)

        for b in range(2):
            pltpu.make_async_copy(rows_v.at[b], out_dst(0), sem_o.at[b]).wait()

    return _sc_gather


_GATHERS = [_make_gather(q) for q in range(NQ)]


def kernel(weight_q, scale, grid):
    wq = weight_q.astype(jnp.int32).reshape(-1)
    grid_rs = grid.reshape(GRID_K * CODESZ // 128, 128)
    tab = _scale_table(scale, grid_rs).reshape(GRID_K, CODESZ)
    parts = [g(tab, wq) for g in _GATHERS]
    rows_per_q = OUT_F // NQ
    return jnp.concatenate(
        [p.reshape(rows_per_q, IN_F) for p in parts], axis=0)


# speedup vs baseline: 50.7417x; 1.2281x over previous
import functools

import jax
import jax.numpy as jnp
from jax import lax
from jax.experimental import pallas as pl
from jax.experimental.pallas import tpu as pltpu
from jax.experimental.pallas import tpu_sc as plsc

OUT_F = 4096
IN_F = 4096
CODESZ = 8
GRID_K = 65536

NC = 2
NS = 16
NW = NC * NS

TOTAL_IDX = OUT_F * (IN_F // CODESZ)
IDX_PER_W = TOTAL_IDX // NW
CHUNK = 2048
N_CHUNKS = IDX_PER_W // CHUNK
NBUF = 3


def _scale_body(s_ref, g_ref, o_ref):
    o_ref[...] = g_ref[...] * s_ref[0]


def _scale_table(scale, grid_rs):
    return pl.pallas_call(
        _scale_body,
        out_shape=jax.ShapeDtypeStruct(grid_rs.shape, jnp.float32),
        in_specs=[
            pl.BlockSpec(memory_space=pltpu.SMEM),
            pl.BlockSpec(memory_space=pltpu.VMEM),
        ],
        out_specs=pl.BlockSpec(memory_space=pltpu.VMEM),
    )(scale, grid_rs)


_MESH = plsc.VectorSubcoreMesh(
    core_axis_name="c", subcore_axis_name="s", num_cores=NC, num_subcores=NS
)


@functools.partial(
    pl.kernel,
    out_type=jax.ShapeDtypeStruct((TOTAL_IDX, CODESZ), jnp.float32),
    mesh=_MESH,
    scratch_types=[
        pltpu.VMEM((NBUF, CHUNK), jnp.int32),
        pltpu.VMEM((NBUF, CHUNK, CODESZ), jnp.float32),
        pltpu.VMEM_SHARED((GRID_K, CODESZ), jnp.float32),
        pltpu.SemaphoreType.DMA,
        pltpu.SemaphoreType.DMA((NBUF,)),
        pltpu.SemaphoreType.DMA((NBUF,)),
    ],
    compiler_params=pltpu.CompilerParams(use_tc_tiling_on_sc=False),
)
def _sc_gather(tab_hbm, wq_hbm, out_hbm, idx_v, rows_v, tab_sp,
               sem_i, sem_g, sem_o):
    s = lax.axis_index("s")
    wid = s * NC + lax.axis_index("c")
    base0 = wid * IDX_PER_W

    tr = GRID_K // NS
    pltpu.sync_copy(tab_hbm.at[pl.ds(s * tr, tr)], tab_sp.at[pl.ds(s * tr, tr)])
    plsc.subcore_barrier()

    def idx_src(k):
        return wq_hbm.at[pl.ds(base0 + k * CHUNK, CHUNK)]

    def out_dst(k):
        return out_hbm.at[pl.ds(base0 + k * CHUNK, CHUNK)]

    for b in range(NBUF):
        pltpu.async_copy(idx_src(b), idx_v.at[b], sem_i)
    pltpu.make_async_copy(idx_src(0), idx_v.at[0], sem_i).wait()
    pltpu.async_copy(tab_sp.at[idx_v.at[0]], rows_v.at[0], sem_g.at[0])

    def body(k0, carry):
        for b in range(NBUF):
            k = NBUF * k0 + b
            b1 = (b + 1) % NBUF

            @pl.when(k + 1 < N_CHUNKS)
            def _():
                pltpu.make_async_copy(idx_src(0), idx_v.at[b1], sem_i).wait()

                @pl.when(k + 1 >= NBUF)
                def _():
                    pltpu.make_async_copy(
                        rows_v.at[b1], out_dst(0), sem_o.at[b1]).wait()

                pltpu.async_copy(
                    tab_sp.at[idx_v.at[b1]], rows_v.at[b1], sem_g.at[b1])

            pltpu.make_async_copy(
                tab_sp.at[idx_v.at[b]], rows_v.at[b], sem_g.at[b]).wait()

            @pl.when(k + NBUF < N_CHUNKS)
            def _():
                pltpu.async_copy(idx_src(k + NBUF), idx_v.at[b], sem_i)

            pltpu.async_copy(rows_v.at[b], out_dst(k), sem_o.at[b])
        return carry

    lax.fori_loop(0, N_CHUNKS // NBUF, body, 0)

    for k in range(N_CHUNKS - N_CHUNKS % NBUF, N_CHUNKS):
        b = k % NBUF
        b1 = (b + 1) % NBUF

        if k + 1 < N_CHUNKS:
            pltpu.make_async_copy(idx_src(0), idx_v.at[b1], sem_i).wait()
            pltpu.make_async_copy(
                rows_v.at[b1], out_dst(0), sem_o.at[b1]).wait()
            pltpu.async_copy(
                tab_sp.at[idx_v.at[b1]], rows_v.at[b1], sem_g.at[b1])

        pltpu.make_async_copy(
            tab_sp.at[idx_v.at[b]], rows_v.at[b], sem_g.at[b]).wait()
        pltpu.async_copy(rows_v.at[b], out_dst(k), sem_o.at[b])

    for b in range(NBUF):
        pltpu.make_async_copy(rows_v.at[b], out_dst(0), sem_o.at[b]).wait()


def kernel(weight_q, scale, grid):
    wq = weight_q.astype(jnp.int32).reshape(-1)
    grid_rs = grid.reshape(GRID_K * CODESZ // 128, 128)
    tab = _scale_table(scale, grid_rs).reshape(GRID_K, CODESZ)
    out = _sc_gather(tab, wq)
    return out.reshape(OUT_F, IN_F)


# speedup vs baseline: 50.9554x; 1.0042x over previous
import functools

import jax
import jax.numpy as jnp
from jax import lax
from jax.experimental import pallas as pl
from jax.experimental.pallas import tpu as pltpu
from jax.experimental.pallas import tpu_sc as plsc

OUT_F = 4096
IN_F = 4096
CODESZ = 8
GRID_K = 65536

NC = 2
NS = 16
NW = NC * NS

TOTAL_IDX = OUT_F * (IN_F // CODESZ)
IDX_PER_W = TOTAL_IDX // NW
CHUNK = 4096
N_CHUNKS = IDX_PER_W // CHUNK
NBUF = 2


def _scale_body(s_ref, g_ref, o_ref):
    o_ref[...] = g_ref[...] * s_ref[0]


def _scale_table(scale, grid_rs):
    return pl.pallas_call(
        _scale_body,
        out_shape=jax.ShapeDtypeStruct(grid_rs.shape, jnp.float32),
        in_specs=[
            pl.BlockSpec(memory_space=pltpu.SMEM),
            pl.BlockSpec(memory_space=pltpu.VMEM),
        ],
        out_specs=pl.BlockSpec(memory_space=pltpu.VMEM),
    )(scale, grid_rs)


_MESH = plsc.VectorSubcoreMesh(
    core_axis_name="c", subcore_axis_name="s", num_cores=NC, num_subcores=NS
)


@functools.partial(
    pl.kernel,
    out_type=jax.ShapeDtypeStruct((TOTAL_IDX, CODESZ), jnp.float32),
    mesh=_MESH,
    scratch_types=[
        pltpu.VMEM((NBUF, CHUNK), jnp.int32),
        pltpu.VMEM((NBUF, CHUNK, CODESZ), jnp.float32),
        pltpu.VMEM_SHARED((GRID_K, CODESZ), jnp.float32),
        pltpu.SemaphoreType.DMA,
        pltpu.SemaphoreType.DMA((NBUF,)),
        pltpu.SemaphoreType.DMA((NBUF,)),
    ],
    compiler_params=pltpu.CompilerParams(use_tc_tiling_on_sc=False),
)
def _sc_gather(tab_hbm, wq_hbm, out_hbm, idx_v, rows_v, tab_sp,
               sem_i, sem_g, sem_o):
    s = lax.axis_index("s")
    wid = s * NC + lax.axis_index("c")
    base0 = wid * IDX_PER_W

    tr = GRID_K // NS
    pltpu.sync_copy(tab_hbm.at[pl.ds(s * tr, tr)], tab_sp.at[pl.ds(s * tr, tr)])
    plsc.subcore_barrier()

    def idx_src(k):
        return wq_hbm.at[pl.ds(base0 + k * CHUNK, CHUNK)]

    def out_dst(k):
        return out_hbm.at[pl.ds(base0 + k * CHUNK, CHUNK)]

    for b in range(NBUF):
        pltpu.async_copy(idx_src(b), idx_v.at[b], sem_i)
    pltpu.make_async_copy(idx_src(0), idx_v.at[0], sem_i).wait()
    pltpu.async_copy(tab_sp.at[idx_v.at[0]], rows_v.at[0], sem_g.at[0])

    def body(k0, carry):
        for b in range(NBUF):
            k = NBUF * k0 + b
            b1 = (b + 1) % NBUF

            @pl.when(k + 1 < N_CHUNKS)
            def _():
                pltpu.make_async_copy(idx_src(0), idx_v.at[b1], sem_i).wait()

                @pl.when(k + 1 >= NBUF)
                def _():
                    pltpu.make_async_copy(
                        rows_v.at[b1], out_dst(0), sem_o.at[b1]).wait()

                pltpu.async_copy(
                    tab_sp.at[idx_v.at[b1]], rows_v.at[b1], sem_g.at[b1])

            pltpu.make_async_copy(
                tab_sp.at[idx_v.at[b]], rows_v.at[b], sem_g.at[b]).wait()

            @pl.when(k + NBUF < N_CHUNKS)
            def _():
                pltpu.async_copy(idx_src(k + NBUF), idx_v.at[b], sem_i)

            pltpu.async_copy(rows_v.at[b], out_dst(k), sem_o.at[b])
        return carry

    lax.fori_loop(0, N_CHUNKS // NBUF, body, 0)

    for k in range(N_CHUNKS - N_CHUNKS % NBUF, N_CHUNKS):
        b = k % NBUF
        b1 = (b + 1) % NBUF

        if k + 1 < N_CHUNKS:
            pltpu.make_async_copy(idx_src(0), idx_v.at[b1], sem_i).wait()
            pltpu.make_async_copy(
                rows_v.at[b1], out_dst(0), sem_o.at[b1]).wait()
            pltpu.async_copy(
                tab_sp.at[idx_v.at[b1]], rows_v.at[b1], sem_g.at[b1])

        pltpu.make_async_copy(
            tab_sp.at[idx_v.at[b]], rows_v.at[b], sem_g.at[b]).wait()
        pltpu.async_copy(rows_v.at[b], out_dst(k), sem_o.at[b])

    for b in range(NBUF):
        pltpu.make_async_copy(rows_v.at[b], out_dst(0), sem_o.at[b]).wait()


def kernel(weight_q, scale, grid):
    wq = weight_q.astype(jnp.int32).reshape(-1)
    grid_rs = grid.reshape(GRID_K * CODESZ // 128, 128)
    tab = _scale_table(scale, grid_rs).reshape(GRID_K, CODESZ)
    out = _sc_gather(tab, wq)
    return out.reshape(OUT_F, IN_F)
